# Initial kernel scaffold; baseline (speedup 1.0000x reference)
import jax
import jax.numpy as jnp
from jax.experimental import pallas as pl


def kernel(x, W_C, W_lamb, conv_w, comb_w, W_P, edge_index):
    raise NotImplementedError("write your pallas kernel here")


# speedup vs baseline: 12.3394x; 12.3394x over previous
import functools

import jax
import jax.numpy as jnp
import numpy as np
from jax import lax
from jax.experimental import pallas as pl
from jax.experimental.pallas import tpu as pltpu
from jax.experimental.pallas import tpu_sc as plsc

N = 10000
E = 320000
IN_C = 128
OUT_C = 128
R_D = 8
R_P = 16
ORDER = 10
RANK = R_D * R_P

NT = 16
NP = 10240
NR = NP // NT
CH = 157
EW = CH * 128
EP = NT * EW

_SC_MESH = dict(core_axis_name="c", subcore_axis_name="s", num_cores=1)


def _sc_deg_body(dst_hbm, zeros_hbm, ones_hbm, deg_hbm,
                 accs, dst_v, zero_v, ones_v):
    sid = lax.axis_index("s")
    row0 = sid * NR

    pltpu.sync_copy(dst_hbm.at[sid], dst_v)
    pltpu.sync_copy(zeros_hbm, zero_v)
    pltpu.sync_copy(ones_hbm, ones_v)

    pltpu.sync_copy(zero_v, accs.at[pl.ds(row0, NR)])
    plsc.subcore_barrier()

    def deg_chunk(j, carry):
        pltpu.sync_copy(ones_v, accs.at[dst_v.at[j]], add=True)
        return carry

    lax.fori_loop(0, CH, deg_chunk, None)
    plsc.subcore_barrier()

    pltpu.sync_copy(accs.at[pl.ds(row0, NR)], deg_hbm.at[pl.ds(row0, NR)])


def _sc_degree(dst3):
    mesh = plsc.VectorSubcoreMesh(**_SC_MESH)
    fn = functools.partial(
        pl.kernel, _sc_deg_body, mesh=mesh,
        compiler_params=pltpu.CompilerParams(use_tc_tiling_on_sc=False),
        out_type=jax.ShapeDtypeStruct((NP, R_P), jnp.float32),
        scratch_types=[
            pltpu.VMEM_SHARED((NP, R_P), jnp.float32),
            pltpu.VMEM((CH, 128), jnp.int32),
            pltpu.VMEM((NR, R_P), jnp.float32),
            pltpu.VMEM((128, R_P), jnp.float32),
        ],
    )()
    zeros = jnp.zeros((NR, R_P), jnp.float32)
    ones = jnp.ones((128, R_P), jnp.float32)
    return fn(dst3, zeros, ones)


def _tc_pre_body(x_ref, b_ref, deg_ref, g_ref, d2_ref, dinv_ref):
    g = jnp.dot(x_ref[...], b_ref[...], preferred_element_type=jnp.float32)
    deg = deg_ref[...]
    dinv = jnp.where(deg > 0.0, lax.rsqrt(jnp.maximum(deg, 1.0)), 0.0)
    for k in range(ORDER):
        g_ref[k] = dinv * g[:, k * R_P:(k + 1) * R_P]
    g_ref[ORDER] = g[:, ORDER * R_P:(ORDER + 1) * R_P]
    d2_ref[...] = dinv * dinv
    dinv_ref[...] = dinv


def _tc_pre(xpad, bprop, degrow):
    bn = 1024
    return pl.pallas_call(
        _tc_pre_body,
        grid=(NP // bn,),
        in_specs=[
            pl.BlockSpec((bn, IN_C), lambda i: (i, 0)),
            pl.BlockSpec((IN_C, (ORDER + 1) * R_P), lambda i: (0, 0)),
            pl.BlockSpec((bn, R_P), lambda i: (i, 0)),
        ],
        out_specs=[
            pl.BlockSpec((ORDER + 1, bn, R_P), lambda i: (0, i, 0)),
            pl.BlockSpec((bn, R_P), lambda i: (i, 0)),
            pl.BlockSpec((bn, R_P), lambda i: (i, 0)),
        ],
        out_shape=[
            jax.ShapeDtypeStruct((ORDER + 1, NP, R_P), jnp.float32),
            jax.ShapeDtypeStruct((NP, R_P), jnp.float32),
            jax.ShapeDtypeStruct((NP, R_P), jnp.float32),
        ],
    )(xpad, bprop, degrow)


def _tc_post_body(t_ref, wp_ref, o_ref):
    o_ref[...] = jnp.dot(t_ref[...], wp_ref[...],
                         preferred_element_type=jnp.float32)


def _tc_post(t, W_P):
    bn = 1000
    return pl.pallas_call(
        _tc_post_body,
        grid=(N // bn,),
        in_specs=[
            pl.BlockSpec((bn, R_P), lambda i: (i, 0)),
            pl.BlockSpec((R_P, OUT_C), lambda i: (0, 0)),
        ],
        out_specs=pl.BlockSpec((bn, OUT_C), lambda i: (i, 0)),
        out_shape=jax.ShapeDtypeStruct((N, OUT_C), jnp.float32),
    )(t, W_P)


def _sc_hops_body(g_hbm, src_hbm, dst_hbm, d2_hbm, dinv_hbm, zeros_hbm,
                  t_hbm, w0_hbm, w1_hbm,
                  accs, src_v, dst_v, rows_v, acc_v, g_v, w_v,
                  dinv_v, d2_v, zero_v):
    sid = lax.axis_index("s")
    row0 = sid * NR

    pltpu.sync_copy(src_hbm.at[sid], src_v)
    pltpu.sync_copy(dst_hbm.at[sid], dst_v)
    pltpu.sync_copy(d2_hbm.at[pl.ds(row0, NR)], d2_v)
    pltpu.sync_copy(dinv_hbm.at[pl.ds(row0, NR)], dinv_v)
    pltpu.sync_copy(zeros_hbm, zero_v)
    plsc.subcore_barrier()

    wbufs = [w0_hbm, w1_hbm]
    for k in range(ORDER):
        w_cur = None if k == 0 else wbufs[(k - 1) % 2]
        w_nxt = wbufs[k % 2]

        pltpu.sync_copy(zero_v, accs.at[pl.ds(row0, NR)])
        plsc.subcore_barrier()

        if k == 0:
            def edge_chunk(j, carry):
                pltpu.sync_copy(g_hbm.at[src_v.at[j]], rows_v)
                pltpu.sync_copy(rows_v, accs.at[dst_v.at[j]], add=True)
                return carry
        else:
            def edge_chunk(j, carry, w_cur=w_cur):
                pltpu.sync_copy(w_cur.at[src_v.at[j]], rows_v)
                pltpu.sync_copy(rows_v, accs.at[dst_v.at[j]], add=True)
                return carry

        lax.fori_loop(0, CH, edge_chunk, None)
        plsc.subcore_barrier()

        pltpu.sync_copy(accs.at[pl.ds(row0, NR)], acc_v)
        pltpu.sync_copy(g_hbm.at[pl.ds((k + 1) * NP + row0, NR)], g_v)

        scl_v = d2_v if k < ORDER - 1 else dinv_v
        out_hbm = w_nxt if k < ORDER - 1 else t_hbm

        def upd_row(r, carry, scl_v=scl_v):
            w_v[r] = g_v[r] + scl_v[r] * acc_v[r]
            return carry

        lax.fori_loop(0, NR, upd_row, None)
        pltpu.sync_copy(w_v, out_hbm.at[pl.ds(row0, NR)])
        plsc.subcore_barrier()


def _sc_propagate(gflat, src3, dst3, d2b, dinvb):
    mesh = plsc.VectorSubcoreMesh(**_SC_MESH)
    fn = functools.partial(
        pl.kernel, _sc_hops_body, mesh=mesh,
        compiler_params=pltpu.CompilerParams(use_tc_tiling_on_sc=False),
        out_type=[
            jax.ShapeDtypeStruct((NP, R_P), jnp.float32),
            jax.ShapeDtypeStruct((NP, R_P), jnp.float32),
            jax.ShapeDtypeStruct((NP, R_P), jnp.float32),
        ],
        scratch_types=[
            pltpu.VMEM_SHARED((NP, R_P), jnp.float32),
            pltpu.VMEM((CH, 128), jnp.int32),
            pltpu.VMEM((CH, 128), jnp.int32),
            pltpu.VMEM((128, R_P), jnp.float32),
            pltpu.VMEM((NR, R_P), jnp.float32),
            pltpu.VMEM((NR, R_P), jnp.float32),
            pltpu.VMEM((NR, R_P), jnp.float32),
            pltpu.VMEM((NR, R_P), jnp.float32),
            pltpu.VMEM((NR, R_P), jnp.float32),
            pltpu.VMEM((NR, R_P), jnp.float32),
        ],
    )()
    zeros = jnp.zeros((NR, R_P), jnp.float32)
    t, _, _ = fn(gflat, src3, dst3, d2b, dinvb, zeros)
    return t


def kernel(x, W_C, W_lamb, conv_w, comb_w, W_P, edge_index):
    a = (conv_w * jnp.tanh(1.0 / (conv_w + 1e-5)))[:, 0, :]
    c = jnp.cumprod(a, axis=0)
    comb = comb_w[0, :, 0, :]
    u = c * jnp.tile(comb, (1, R_P))
    kk = np.arange(RANK)
    sel = jnp.asarray((kk[:, None] // R_D) ==
                      np.arange(R_P)[None, :], jnp.float32)
    ublocks = [u[ORDER - k][:, None] * sel for k in range(ORDER + 1)]
    umat = jnp.concatenate(ublocks, axis=1)
    bprop = W_C @ W_lamb @ umat

    src = edge_index[0]
    dst = edge_index[1]
    pad = jnp.full((EP - E,), NP - 1, jnp.int32)
    src3 = jnp.concatenate([src, pad]).reshape(NT, CH, 128)
    dst3 = jnp.concatenate([dst, pad]).reshape(NT, CH, 128)

    degrow = _sc_degree(dst3)

    xpad = jnp.pad(x, ((0, NP - N), (0, 0)))
    gprop, d2b, dinvb = _tc_pre(xpad, bprop, degrow)
    gflat = gprop.reshape((ORDER + 1) * NP, R_P)

    t = _sc_propagate(gflat, src3, dst3, d2b, dinvb)
    return _tc_post(t[:N], W_P)


# speedup vs baseline: 26.0015x; 2.1072x over previous
import functools

import jax
import jax.numpy as jnp
import numpy as np
from jax import lax
from jax.experimental import pallas as pl
from jax.experimental.pallas import tpu as pltpu
from jax.experimental.pallas import tpu_sc as plsc

N = 10000
E = 320000
IN_C = 128
OUT_C = 128
R_D = 8
R_P = 16
ORDER = 10
RANK = R_D * R_P

NT = 16
NP = 10240
NR = NP // NT
CH = 157
EW = CH * 128
EP = NT * EW

_SC_MESH = dict(core_axis_name="c", subcore_axis_name="s", num_cores=1)

KB = 4
NS = 2 * KB


def _sc_deg_body(dst_hbm, zeros_hbm, ones_hbm, deg_hbm,
                 accs, dst_v, zero_v, ones_v, ssem):
    sid = lax.axis_index("s")
    row0 = sid * NR

    pltpu.sync_copy(dst_hbm.at[sid], dst_v)
    pltpu.sync_copy(zeros_hbm, zero_v)
    pltpu.sync_copy(ones_hbm, ones_v)

    pltpu.sync_copy(zero_v, accs.at[pl.ds(row0, NR)])
    plsc.subcore_barrier()

    def deg_chunk(j, carry):
        pltpu.async_copy(ones_v, accs.at[dst_v.at[j]], ssem, add=True)

        @pl.when(j >= KB)
        def _drain():
            pltpu.make_async_copy(ones_v, accs.at[dst_v.at[j - KB]],
                                  ssem).wait()

        return carry

    lax.fori_loop(0, CH, deg_chunk, None)
    for b in range(KB):
        jd = CH - KB + b
        pltpu.make_async_copy(ones_v, accs.at[dst_v.at[jd]], ssem).wait()
    plsc.subcore_barrier()

    pltpu.sync_copy(accs.at[pl.ds(row0, NR)], deg_hbm.at[pl.ds(row0, NR)])


def _sc_degree(dst3):
    mesh = plsc.VectorSubcoreMesh(**_SC_MESH)
    fn = functools.partial(
        pl.kernel, _sc_deg_body, mesh=mesh,
        compiler_params=pltpu.CompilerParams(use_tc_tiling_on_sc=False),
        out_type=jax.ShapeDtypeStruct((NP, R_P), jnp.float32),
        scratch_types=[
            pltpu.VMEM_SHARED((NP, R_P), jnp.float32),
            pltpu.VMEM((CH, 128), jnp.int32),
            pltpu.VMEM((NR, R_P), jnp.float32),
            pltpu.VMEM((128, R_P), jnp.float32),
            pltpu.SemaphoreType.DMA,
        ],
    )()
    zeros = jnp.zeros((NR, R_P), jnp.float32)
    ones = jnp.ones((128, R_P), jnp.float32)
    return fn(dst3, zeros, ones)


def _tc_pre_body(x_ref, b_ref, deg_ref, g_ref, d2_ref, dinv_ref):
    g = jnp.dot(x_ref[...], b_ref[...], preferred_element_type=jnp.float32)
    deg = deg_ref[...]
    dinv = jnp.where(deg > 0.0, lax.rsqrt(jnp.maximum(deg, 1.0)), 0.0)
    for k in range(ORDER):
        g_ref[k] = dinv * g[:, k * R_P:(k + 1) * R_P]
    g_ref[ORDER] = g[:, ORDER * R_P:(ORDER + 1) * R_P]
    d2_ref[...] = dinv * dinv
    dinv_ref[...] = dinv


def _tc_pre(xpad, bprop, degrow):
    bn = 1024
    return pl.pallas_call(
        _tc_pre_body,
        grid=(NP // bn,),
        in_specs=[
            pl.BlockSpec((bn, IN_C), lambda i: (i, 0)),
            pl.BlockSpec((IN_C, (ORDER + 1) * R_P), lambda i: (0, 0)),
            pl.BlockSpec((bn, R_P), lambda i: (i, 0)),
        ],
        out_specs=[
            pl.BlockSpec((ORDER + 1, bn, R_P), lambda i: (0, i, 0)),
            pl.BlockSpec((bn, R_P), lambda i: (i, 0)),
            pl.BlockSpec((bn, R_P), lambda i: (i, 0)),
        ],
        out_shape=[
            jax.ShapeDtypeStruct((ORDER + 1, NP, R_P), jnp.float32),
            jax.ShapeDtypeStruct((NP, R_P), jnp.float32),
            jax.ShapeDtypeStruct((NP, R_P), jnp.float32),
        ],
    )(xpad, bprop, degrow)


def _tc_post_body(t_ref, wp_ref, o_ref):
    o_ref[...] = jnp.dot(t_ref[...], wp_ref[...],
                         preferred_element_type=jnp.float32)


def _tc_post(t, W_P):
    bn = 1000
    return pl.pallas_call(
        _tc_post_body,
        grid=(N // bn,),
        in_specs=[
            pl.BlockSpec((bn, R_P), lambda i: (i, 0)),
            pl.BlockSpec((R_P, OUT_C), lambda i: (0, 0)),
        ],
        out_specs=pl.BlockSpec((bn, OUT_C), lambda i: (i, 0)),
        out_shape=jax.ShapeDtypeStruct((N, OUT_C), jnp.float32),
    )(t, W_P)


def _sc_hops_body(g_hbm, src_hbm, dst_hbm, d2_hbm, dinv_hbm, zeros_hbm,
                  t_hbm, w0_hbm, w1_hbm,
                  accs, src_v, dst_v, rows_v, acc_v, g_v, w_v,
                  dinv_v, d2_v, zero_v, gsem, ssem):
    sid = lax.axis_index("s")
    row0 = sid * NR

    pltpu.sync_copy(src_hbm.at[sid], src_v)
    pltpu.sync_copy(dst_hbm.at[sid], dst_v)
    pltpu.sync_copy(d2_hbm.at[pl.ds(row0, NR)], d2_v)
    pltpu.sync_copy(dinv_hbm.at[pl.ds(row0, NR)], dinv_v)
    pltpu.sync_copy(zeros_hbm, zero_v)
    plsc.subcore_barrier()

    wbufs = [w0_hbm, w1_hbm]
    for k in range(ORDER):
        w_cur = None if k == 0 else wbufs[(k - 1) % 2]
        w_nxt = wbufs[k % 2]

        pltpu.sync_copy(zero_v, accs.at[pl.ds(row0, NR)])
        plsc.subcore_barrier()

        w_gat = g_hbm if k == 0 else w_cur

        for b in range(KB):
            pltpu.async_copy(w_gat.at[src_v.at[b]], rows_v.at[b], gsem)

        def edge_chunk(j, carry, w_gat=w_gat):
            slot = lax.rem(j, NS)
            pltpu.make_async_copy(w_gat.at[src_v.at[j]], rows_v.at[slot],
                                  gsem).wait()
            pltpu.async_copy(rows_v.at[slot], accs.at[dst_v.at[j]], ssem,
                             add=True)

            @pl.when(j >= KB)
            def _drain():
                jd = j - KB
                pltpu.make_async_copy(rows_v.at[lax.rem(jd, NS)],
                                      accs.at[dst_v.at[jd]], ssem).wait()

            @pl.when(j + KB < CH)
            def _prefetch():
                jg = j + KB
                pltpu.async_copy(w_gat.at[src_v.at[jg]],
                                 rows_v.at[lax.rem(jg, NS)], gsem)

            return carry

        lax.fori_loop(0, CH, edge_chunk, None)
        for b in range(KB):
            jd = CH - KB + b
            pltpu.make_async_copy(rows_v.at[jd % NS], accs.at[dst_v.at[jd]],
                                  ssem).wait()
        plsc.subcore_barrier()

        pltpu.sync_copy(accs.at[pl.ds(row0, NR)], acc_v)
        pltpu.sync_copy(g_hbm.at[pl.ds((k + 1) * NP + row0, NR)], g_v)

        scl_v = d2_v if k < ORDER - 1 else dinv_v
        out_hbm = w_nxt if k < ORDER - 1 else t_hbm

        def upd_row(r, carry, scl_v=scl_v):
            w_v[r] = g_v[r] + scl_v[r] * acc_v[r]
            return carry

        lax.fori_loop(0, NR, upd_row, None)
        pltpu.sync_copy(w_v, out_hbm.at[pl.ds(row0, NR)])
        plsc.subcore_barrier()


def _sc_propagate(gflat, src3, dst3, d2b, dinvb):
    mesh = plsc.VectorSubcoreMesh(**_SC_MESH)
    fn = functools.partial(
        pl.kernel, _sc_hops_body, mesh=mesh,
        compiler_params=pltpu.CompilerParams(use_tc_tiling_on_sc=False),
        out_type=[
            jax.ShapeDtypeStruct((NP, R_P), jnp.float32),
            jax.ShapeDtypeStruct((NP, R_P), jnp.float32),
            jax.ShapeDtypeStruct((NP, R_P), jnp.float32),
        ],
        scratch_types=[
            pltpu.VMEM_SHARED((NP, R_P), jnp.float32),
            pltpu.VMEM((CH, 128), jnp.int32),
            pltpu.VMEM((CH, 128), jnp.int32),
            pltpu.VMEM((NS, 128, R_P), jnp.float32),
            pltpu.VMEM((NR, R_P), jnp.float32),
            pltpu.VMEM((NR, R_P), jnp.float32),
            pltpu.VMEM((NR, R_P), jnp.float32),
            pltpu.VMEM((NR, R_P), jnp.float32),
            pltpu.VMEM((NR, R_P), jnp.float32),
            pltpu.VMEM((NR, R_P), jnp.float32),
            pltpu.SemaphoreType.DMA,
            pltpu.SemaphoreType.DMA,
        ],
    )()
    zeros = jnp.zeros((NR, R_P), jnp.float32)
    t, _, _ = fn(gflat, src3, dst3, d2b, dinvb, zeros)
    return t


def kernel(x, W_C, W_lamb, conv_w, comb_w, W_P, edge_index):
    a = (conv_w * jnp.tanh(1.0 / (conv_w + 1e-5)))[:, 0, :]
    c = jnp.cumprod(a, axis=0)
    comb = comb_w[0, :, 0, :]
    u = c * jnp.tile(comb, (1, R_P))
    kk = np.arange(RANK)
    sel = jnp.asarray((kk[:, None] // R_D) ==
                      np.arange(R_P)[None, :], jnp.float32)
    ublocks = [u[ORDER - k][:, None] * sel for k in range(ORDER + 1)]
    umat = jnp.concatenate(ublocks, axis=1)
    bprop = W_C @ W_lamb @ umat

    src = edge_index[0]
    dst = edge_index[1]
    pad = jnp.full((EP - E,), NP - 1, jnp.int32)
    src3 = jnp.concatenate([src, pad]).reshape(NT, CH, 128)
    dst3 = jnp.concatenate([dst, pad]).reshape(NT, CH, 128)

    degrow = _sc_degree(dst3)

    xpad = jnp.pad(x, ((0, NP - N), (0, 0)))
    gprop, d2b, dinvb = _tc_pre(xpad, bprop, degrow)
    gflat = gprop.reshape((ORDER + 1) * NP, R_P)

    t = _sc_propagate(gflat, src3, dst3, d2b, dinvb)
    return _tc_post(t[:N], W_P)


# speedup vs baseline: 30.6886x; 1.1803x over previous
import functools

import jax
import jax.numpy as jnp
import numpy as np
from jax import lax
from jax.experimental import pallas as pl
from jax.experimental.pallas import tpu as pltpu
from jax.experimental.pallas import tpu_sc as plsc

N = 10000
E = 320000
IN_C = 128
OUT_C = 128
R_D = 8
R_P = 16
ORDER = 10
RANK = R_D * R_P

NT = 16
NP = 10240
NR = NP // NT
CH = 157
EW = CH * 128
EP = NT * EW

_SC_MESH = dict(core_axis_name="c", subcore_axis_name="s", num_cores=1)

KB = 6
NS = 2 * KB


def _sc_deg_body(dst_hbm, zeros_hbm, ones_hbm, deg_hbm,
                 accs, dst_v, zero_v, ones_v, ssem):
    sid = lax.axis_index("s")
    row0 = sid * NR

    pltpu.sync_copy(dst_hbm.at[sid], dst_v)
    pltpu.sync_copy(zeros_hbm, zero_v)
    pltpu.sync_copy(ones_hbm, ones_v)

    pltpu.sync_copy(zero_v, accs.at[pl.ds(row0, NR)])
    plsc.subcore_barrier()

    def deg_chunk(j, carry):
        pltpu.async_copy(ones_v, accs.at[dst_v.at[j]], ssem, add=True)

        @pl.when(j >= KB)
        def _drain():
            pltpu.make_async_copy(ones_v, accs.at[dst_v.at[j - KB]],
                                  ssem).wait()

        return carry

    lax.fori_loop(0, CH, deg_chunk, None)
    for b in range(KB):
        jd = CH - KB + b
        pltpu.make_async_copy(ones_v, accs.at[dst_v.at[jd]], ssem).wait()
    plsc.subcore_barrier()

    pltpu.sync_copy(accs.at[pl.ds(row0, NR)], deg_hbm.at[pl.ds(row0, NR)])


def _sc_degree(dst3):
    mesh = plsc.VectorSubcoreMesh(**_SC_MESH)
    fn = functools.partial(
        pl.kernel, _sc_deg_body, mesh=mesh,
        compiler_params=pltpu.CompilerParams(use_tc_tiling_on_sc=False),
        out_type=jax.ShapeDtypeStruct((NP, R_P), jnp.float32),
        scratch_types=[
            pltpu.VMEM_SHARED((NP, R_P), jnp.float32),
            pltpu.VMEM((CH, 128), jnp.int32),
            pltpu.VMEM((NR, R_P), jnp.float32),
            pltpu.VMEM((128, R_P), jnp.float32),
            pltpu.SemaphoreType.DMA,
        ],
    )()
    zeros = jnp.zeros((NR, R_P), jnp.float32)
    ones = jnp.ones((128, R_P), jnp.float32)
    return fn(dst3, zeros, ones)


def _tc_pre_body(x_ref, b_ref, deg_ref, g_ref, d2_ref, dinv_ref):
    g = jnp.dot(x_ref[...], b_ref[...], preferred_element_type=jnp.float32)
    deg = deg_ref[...]
    dinv = jnp.where(deg > 0.0, lax.rsqrt(jnp.maximum(deg, 1.0)), 0.0)
    for k in range(ORDER):
        g_ref[k] = dinv * g[:, k * R_P:(k + 1) * R_P]
    g_ref[ORDER] = g[:, ORDER * R_P:(ORDER + 1) * R_P]
    d2_ref[...] = dinv * dinv
    dinv_ref[...] = dinv


def _tc_pre(xpad, bprop, degrow):
    bn = 1024
    return pl.pallas_call(
        _tc_pre_body,
        grid=(NP // bn,),
        in_specs=[
            pl.BlockSpec((bn, IN_C), lambda i: (i, 0)),
            pl.BlockSpec((IN_C, (ORDER + 1) * R_P), lambda i: (0, 0)),
            pl.BlockSpec((bn, R_P), lambda i: (i, 0)),
        ],
        out_specs=[
            pl.BlockSpec((ORDER + 1, bn, R_P), lambda i: (0, i, 0)),
            pl.BlockSpec((bn, R_P), lambda i: (i, 0)),
            pl.BlockSpec((bn, R_P), lambda i: (i, 0)),
        ],
        out_shape=[
            jax.ShapeDtypeStruct((ORDER + 1, NP, R_P), jnp.float32),
            jax.ShapeDtypeStruct((NP, R_P), jnp.float32),
            jax.ShapeDtypeStruct((NP, R_P), jnp.float32),
        ],
    )(xpad, bprop, degrow)


def _tc_post_body(t_ref, wp_ref, o_ref):
    o_ref[...] = jnp.dot(t_ref[...], wp_ref[...],
                         preferred_element_type=jnp.float32)


def _tc_post(t, W_P):
    bn = 1000
    return pl.pallas_call(
        _tc_post_body,
        grid=(N // bn,),
        in_specs=[
            pl.BlockSpec((bn, R_P), lambda i: (i, 0)),
            pl.BlockSpec((R_P, OUT_C), lambda i: (0, 0)),
        ],
        out_specs=pl.BlockSpec((bn, OUT_C), lambda i: (i, 0)),
        out_shape=jax.ShapeDtypeStruct((N, OUT_C), jnp.float32),
    )(t, W_P)


def _sc_hops_body(g_hbm, src_hbm, dst_hbm, d2_hbm, dinv_hbm, zeros_hbm,
                  t_hbm, w0_hbm, w1_hbm,
                  accs, src_v, dst_v, rows_v, acc_v, g_v,
                  dinv_v, d2_v, zero_v, gsem, ssem):
    sid = lax.axis_index("s")
    row0 = sid * NR

    pltpu.sync_copy(src_hbm.at[sid], src_v)
    pltpu.sync_copy(dst_hbm.at[sid], dst_v)
    pltpu.sync_copy(d2_hbm.at[pl.ds(row0, NR)], d2_v)
    pltpu.sync_copy(dinv_hbm.at[pl.ds(row0, NR)], dinv_v)
    pltpu.sync_copy(zeros_hbm, zero_v)
    plsc.subcore_barrier()

    wbufs = [w0_hbm, w1_hbm]
    for k in range(ORDER):
        w_cur = None if k == 0 else wbufs[(k - 1) % 2]
        w_nxt = wbufs[k % 2]

        pltpu.sync_copy(zero_v, accs.at[pl.ds(row0, NR)])
        plsc.subcore_barrier()

        w_gat = g_hbm if k == 0 else w_cur

        for b in range(KB):
            pltpu.async_copy(w_gat.at[src_v.at[b]], rows_v.at[b], gsem)

        def edge_chunk(j, carry, w_gat=w_gat):
            slot = lax.rem(j, NS)
            pltpu.make_async_copy(w_gat.at[src_v.at[j]], rows_v.at[slot],
                                  gsem).wait()
            pltpu.async_copy(rows_v.at[slot], accs.at[dst_v.at[j]], ssem,
                             add=True)

            @pl.when(j >= KB)
            def _drain():
                jd = j - KB
                pltpu.make_async_copy(rows_v.at[lax.rem(jd, NS)],
                                      accs.at[dst_v.at[jd]], ssem).wait()

            @pl.when(j + KB < CH)
            def _prefetch():
                jg = j + KB
                pltpu.async_copy(w_gat.at[src_v.at[jg]],
                                 rows_v.at[lax.rem(jg, NS)], gsem)

            return carry

        lax.fori_loop(0, CH, edge_chunk, None)
        for b in range(KB):
            jd = CH - KB + b
            pltpu.make_async_copy(rows_v.at[jd % NS], accs.at[dst_v.at[jd]],
                                  ssem).wait()
        plsc.subcore_barrier()

        pltpu.sync_copy(accs.at[pl.ds(row0, NR)], acc_v)
        pltpu.sync_copy(g_hbm.at[pl.ds((k + 1) * NP + row0, NR)], g_v)

        scl_v = d2_v if k < ORDER - 1 else dinv_v
        out_hbm = w_nxt if k < ORDER - 1 else t_hbm

        def upd_row(r, carry, scl_v=scl_v):
            g_v[r] = g_v[r] + scl_v[r] * acc_v[r]
            return carry

        lax.fori_loop(0, NR, upd_row, None)
        pltpu.sync_copy(g_v, out_hbm.at[pl.ds(row0, NR)])
        plsc.subcore_barrier()


def _sc_propagate(gflat, src3, dst3, d2b, dinvb):
    mesh = plsc.VectorSubcoreMesh(**_SC_MESH)
    fn = functools.partial(
        pl.kernel, _sc_hops_body, mesh=mesh,
        compiler_params=pltpu.CompilerParams(use_tc_tiling_on_sc=False),
        out_type=[
            jax.ShapeDtypeStruct((NP, R_P), jnp.float32),
            jax.ShapeDtypeStruct((NP, R_P), jnp.float32),
            jax.ShapeDtypeStruct((NP, R_P), jnp.float32),
        ],
        scratch_types=[
            pltpu.VMEM_SHARED((NP, R_P), jnp.float32),
            pltpu.VMEM((CH, 128), jnp.int32),
            pltpu.VMEM((CH, 128), jnp.int32),
            pltpu.VMEM((NS, 128, R_P), jnp.float32),
            pltpu.VMEM((NR, R_P), jnp.float32),
            pltpu.VMEM((NR, R_P), jnp.float32),
            pltpu.VMEM((NR, R_P), jnp.float32),
            pltpu.VMEM((NR, R_P), jnp.float32),
            pltpu.VMEM((NR, R_P), jnp.float32),
            pltpu.SemaphoreType.DMA,
            pltpu.SemaphoreType.DMA,
        ],
    )()
    zeros = jnp.zeros((NR, R_P), jnp.float32)
    t, _, _ = fn(gflat, src3, dst3, d2b, dinvb, zeros)
    return t


def kernel(x, W_C, W_lamb, conv_w, comb_w, W_P, edge_index):
    a = (conv_w * jnp.tanh(1.0 / (conv_w + 1e-5)))[:, 0, :]
    c = jnp.cumprod(a, axis=0)
    comb = comb_w[0, :, 0, :]
    u = c * jnp.tile(comb, (1, R_P))
    kk = np.arange(RANK)
    sel = jnp.asarray((kk[:, None] // R_D) ==
                      np.arange(R_P)[None, :], jnp.float32)
    ublocks = [u[ORDER - k][:, None] * sel for k in range(ORDER + 1)]
    umat = jnp.concatenate(ublocks, axis=1)
    bprop = W_C @ W_lamb @ umat

    src = edge_index[0]
    dst = edge_index[1]
    pad = jnp.full((EP - E,), NP - 1, jnp.int32)
    src3 = jnp.concatenate([src, pad]).reshape(NT, CH, 128)
    dst3 = jnp.concatenate([dst, pad]).reshape(NT, CH, 128)

    degrow = _sc_degree(dst3)

    xpad = jnp.pad(x, ((0, NP - N), (0, 0)))
    gprop, d2b, dinvb = _tc_pre(xpad, bprop, degrow)
    gflat = gprop.reshape((ORDER + 1) * NP, R_P)

    t = _sc_propagate(gflat, src3, dst3, d2b, dinvb)
    return _tc_post(t[:N], W_P)


# speedup vs baseline: 31.9172x; 1.0400x over previous
import functools

import jax
import jax.numpy as jnp
import numpy as np
from jax import lax
from jax.experimental import pallas as pl
from jax.experimental.pallas import tpu as pltpu
from jax.experimental.pallas import tpu_sc as plsc

N = 10000
E = 320000
IN_C = 128
OUT_C = 128
R_D = 8
R_P = 16
ORDER = 10
RANK = R_D * R_P

NT = 16
NP = 10240
NR = NP // NT
CH = 157
EW = CH * 128
EP = NT * EW

_SC_MESH = dict(core_axis_name="c", subcore_axis_name="s", num_cores=1)

KB = 6
NS = 2 * KB


def _sc_deg_body(dst_hbm, zeros_hbm, ones_hbm, deg_hbm,
                 accs, dst_v, zero_v, ones_v, ssem):
    sid = lax.axis_index("s")
    row0 = sid * NR

    pltpu.sync_copy(dst_hbm.at[sid], dst_v)
    pltpu.sync_copy(zeros_hbm, zero_v)
    pltpu.sync_copy(ones_hbm, ones_v)

    pltpu.sync_copy(zero_v, accs.at[pl.ds(row0, NR)])
    plsc.subcore_barrier()

    def deg_chunk(j, carry):
        pltpu.async_copy(ones_v, accs.at[dst_v.at[j]], ssem, add=True)

        @pl.when(j >= KB)
        def _drain():
            pltpu.make_async_copy(ones_v, accs.at[dst_v.at[j - KB]],
                                  ssem).wait()

        return carry

    lax.fori_loop(0, CH, deg_chunk, None)
    for b in range(KB):
        jd = CH - KB + b
        pltpu.make_async_copy(ones_v, accs.at[dst_v.at[jd]], ssem).wait()
    plsc.subcore_barrier()

    pltpu.sync_copy(accs.at[pl.ds(row0, NR)], deg_hbm.at[pl.ds(row0, NR)])


def _sc_degree(dst3):
    mesh = plsc.VectorSubcoreMesh(**_SC_MESH)
    fn = functools.partial(
        pl.kernel, _sc_deg_body, mesh=mesh,
        compiler_params=pltpu.CompilerParams(use_tc_tiling_on_sc=False),
        out_type=jax.ShapeDtypeStruct((NP, R_P), jnp.float32),
        scratch_types=[
            pltpu.VMEM_SHARED((NP, R_P), jnp.float32),
            pltpu.VMEM((CH, 128), jnp.int32),
            pltpu.VMEM((NR, R_P), jnp.float32),
            pltpu.VMEM((128, R_P), jnp.float32),
            pltpu.SemaphoreType.DMA,
        ],
    )()
    zeros = jnp.zeros((NR, R_P), jnp.float32)
    ones = jnp.ones((128, R_P), jnp.float32)
    return fn(dst3, zeros, ones)


def _tc_pre_body(x_ref, b_ref, g_ref):
    g = jnp.dot(x_ref[...], b_ref[...], preferred_element_type=jnp.float32)
    for k in range(ORDER + 1):
        g_ref[k] = g[:, k * R_P:(k + 1) * R_P]


def _tc_pre(xpad, bprop):
    bn = 1024
    return pl.pallas_call(
        _tc_pre_body,
        grid=(NP // bn,),
        in_specs=[
            pl.BlockSpec((bn, IN_C), lambda i: (i, 0)),
            pl.BlockSpec((IN_C, (ORDER + 1) * R_P), lambda i: (0, 0)),
        ],
        out_specs=pl.BlockSpec((ORDER + 1, bn, R_P), lambda i: (0, i, 0)),
        out_shape=jax.ShapeDtypeStruct((ORDER + 1, NP, R_P), jnp.float32),
    )(xpad, bprop)


def _tc_dinv_body(deg_ref, dinv_ref, d2_ref):
    deg = deg_ref[...]
    dinv = jnp.where(deg > 0.0, lax.rsqrt(jnp.maximum(deg, 1.0)), 0.0)
    dinv_ref[...] = dinv
    d2_ref[...] = dinv * dinv


def _tc_dinv(degrow):
    return pl.pallas_call(
        _tc_dinv_body,
        out_shape=[
            jax.ShapeDtypeStruct((NP, R_P), jnp.float32),
            jax.ShapeDtypeStruct((NP, R_P), jnp.float32),
        ],
    )(degrow)


def _tc_post_body(t_ref, wp_ref, o_ref):
    o_ref[...] = jnp.dot(t_ref[...], wp_ref[...],
                         preferred_element_type=jnp.float32)


def _tc_post(t, W_P):
    bn = 1000
    return pl.pallas_call(
        _tc_post_body,
        grid=(N // bn,),
        in_specs=[
            pl.BlockSpec((bn, R_P), lambda i: (i, 0)),
            pl.BlockSpec((R_P, OUT_C), lambda i: (0, 0)),
        ],
        out_specs=pl.BlockSpec((bn, OUT_C), lambda i: (i, 0)),
        out_shape=jax.ShapeDtypeStruct((N, OUT_C), jnp.float32),
    )(t, W_P)


def _sc_hops_body(g_hbm, src_hbm, dst_hbm, d2_hbm, dinv_hbm, zeros_hbm,
                  t_hbm, w0_hbm, w1_hbm,
                  accs, src_v, dst_v, rows_v, acc_v, g_v,
                  dinv_v, d2_v, zero_v, gsem, ssem):
    sid = lax.axis_index("s")
    row0 = sid * NR

    pltpu.sync_copy(src_hbm.at[sid], src_v)
    pltpu.sync_copy(dst_hbm.at[sid], dst_v)
    pltpu.sync_copy(d2_hbm.at[pl.ds(row0, NR)], d2_v)
    pltpu.sync_copy(dinv_hbm.at[pl.ds(row0, NR)], dinv_v)
    pltpu.sync_copy(zeros_hbm, zero_v)

    pltpu.sync_copy(g_hbm.at[pl.ds(row0, NR)], g_v)

    def v0_row(r, carry):
        g_v[r] = dinv_v[r] * g_v[r]
        return carry

    lax.fori_loop(0, NR, v0_row, None)
    pltpu.sync_copy(g_v, w0_hbm.at[pl.ds(row0, NR)])
    plsc.subcore_barrier()

    wbufs = [w0_hbm, w1_hbm]
    for k in range(ORDER):
        w_nxt = wbufs[k % 2]

        pltpu.sync_copy(zero_v, accs.at[pl.ds(row0, NR)])
        plsc.subcore_barrier()

        w_gat = wbufs[0] if k == 0 else wbufs[(k - 1) % 2]

        for b in range(KB):
            pltpu.async_copy(w_gat.at[src_v.at[b]], rows_v.at[b], gsem)

        def edge_chunk(j, carry, w_gat=w_gat):
            slot = lax.rem(j, NS)
            pltpu.make_async_copy(w_gat.at[src_v.at[j]], rows_v.at[slot],
                                  gsem).wait()
            pltpu.async_copy(rows_v.at[slot], accs.at[dst_v.at[j]], ssem,
                             add=True)

            @pl.when(j >= KB)
            def _drain():
                jd = j - KB
                pltpu.make_async_copy(rows_v.at[lax.rem(jd, NS)],
                                      accs.at[dst_v.at[jd]], ssem).wait()

            @pl.when(j + KB < CH)
            def _prefetch():
                jg = j + KB
                pltpu.async_copy(w_gat.at[src_v.at[jg]],
                                 rows_v.at[lax.rem(jg, NS)], gsem)

            return carry

        lax.fori_loop(0, CH, edge_chunk, None)
        for b in range(KB):
            jd = CH - KB + b
            pltpu.make_async_copy(rows_v.at[jd % NS], accs.at[dst_v.at[jd]],
                                  ssem).wait()
        plsc.subcore_barrier()

        pltpu.sync_copy(accs.at[pl.ds(row0, NR)], acc_v)
        pltpu.sync_copy(g_hbm.at[pl.ds((k + 1) * NP + row0, NR)], g_v)

        out_hbm = w_nxt if k < ORDER - 1 else t_hbm

        if k < ORDER - 1:
            def upd_row(r, carry):
                g_v[r] = dinv_v[r] * g_v[r] + d2_v[r] * acc_v[r]
                return carry
        else:
            def upd_row(r, carry):
                g_v[r] = g_v[r] + dinv_v[r] * acc_v[r]
                return carry

        lax.fori_loop(0, NR, upd_row, None)
        pltpu.sync_copy(g_v, out_hbm.at[pl.ds(row0, NR)])
        plsc.subcore_barrier()


def _sc_propagate(gflat, src3, dst3, d2b, dinvb):
    mesh = plsc.VectorSubcoreMesh(**_SC_MESH)
    fn = functools.partial(
        pl.kernel, _sc_hops_body, mesh=mesh,
        compiler_params=pltpu.CompilerParams(use_tc_tiling_on_sc=False),
        out_type=[
            jax.ShapeDtypeStruct((NP, R_P), jnp.float32),
            jax.ShapeDtypeStruct((NP, R_P), jnp.float32),
            jax.ShapeDtypeStruct((NP, R_P), jnp.float32),
        ],
        scratch_types=[
            pltpu.VMEM_SHARED((NP, R_P), jnp.float32),
            pltpu.VMEM((CH, 128), jnp.int32),
            pltpu.VMEM((CH, 128), jnp.int32),
            pltpu.VMEM((NS, 128, R_P), jnp.float32),
            pltpu.VMEM((NR, R_P), jnp.float32),
            pltpu.VMEM((NR, R_P), jnp.float32),
            pltpu.VMEM((NR, R_P), jnp.float32),
            pltpu.VMEM((NR, R_P), jnp.float32),
            pltpu.VMEM((NR, R_P), jnp.float32),
            pltpu.SemaphoreType.DMA,
            pltpu.SemaphoreType.DMA,
        ],
    )()
    zeros = jnp.zeros((NR, R_P), jnp.float32)
    t, _, _ = fn(gflat, src3, dst3, d2b, dinvb, zeros)
    return t


def kernel(x, W_C, W_lamb, conv_w, comb_w, W_P, edge_index):
    a = (conv_w * jnp.tanh(1.0 / (conv_w + 1e-5)))[:, 0, :]
    c = jnp.cumprod(a, axis=0)
    comb = comb_w[0, :, 0, :]
    u = c * jnp.tile(comb, (1, R_P))
    kk = np.arange(RANK)
    sel = jnp.asarray((kk[:, None] // R_D) ==
                      np.arange(R_P)[None, :], jnp.float32)
    ublocks = [u[ORDER - k][:, None] * sel for k in range(ORDER + 1)]
    umat = jnp.concatenate(ublocks, axis=1)
    bprop = W_C @ W_lamb @ umat

    src = edge_index[0]
    dst = edge_index[1]
    pad = jnp.full((EP - E,), NP - 1, jnp.int32)
    src3 = jnp.concatenate([src, pad]).reshape(NT, CH, 128)
    dst3 = jnp.concatenate([dst, pad]).reshape(NT, CH, 128)

    degrow = _sc_degree(dst3)
    xpad = jnp.pad(x, ((0, NP - N), (0, 0)))
    gprop = _tc_pre(xpad, bprop)
    gflat = gprop.reshape((ORDER + 1) * NP, R_P)
    dinvb, d2b = _tc_dinv(degrow)

    t = _sc_propagate(gflat, src3, dst3, d2b, dinvb)
    return _tc_post(t[:N], W_P)


# speedup vs baseline: 33.7971x; 1.0589x over previous
import functools

import jax
import jax.numpy as jnp
import numpy as np
from jax import lax
from jax.experimental import pallas as pl
from jax.experimental.pallas import tpu as pltpu
from jax.experimental.pallas import tpu_sc as plsc

N = 10000
E = 320000
IN_C = 128
OUT_C = 128
R_D = 8
R_P = 16
ORDER = 10
RANK = R_D * R_P

NT = 16
NP = 10240
NR = NP // NT
CH = 157
EW = CH * 128
EP = NT * EW

_SC_MESH = dict(core_axis_name="c", subcore_axis_name="s", num_cores=1)

KB = 7
NS = 2 * KB


def _sc_deg_body(dst_hbm, zeros_hbm, ones_hbm, deg_hbm,
                 accs, dst_v, zero_v, ones_v, ssem):
    sid = lax.axis_index("s")
    row0 = sid * NR

    pltpu.sync_copy(dst_hbm.at[sid], dst_v)
    pltpu.sync_copy(zeros_hbm, zero_v)
    pltpu.sync_copy(ones_hbm, ones_v)

    pltpu.sync_copy(zero_v, accs.at[pl.ds(row0, NR)])
    plsc.subcore_barrier()

    def deg_chunk(j, carry):
        pltpu.async_copy(ones_v, accs.at[dst_v.at[j]], ssem, add=True)

        @pl.when(j >= KB)
        def _drain():
            pltpu.make_async_copy(ones_v, accs.at[dst_v.at[j - KB]],
                                  ssem).wait()

        return carry

    lax.fori_loop(0, CH, deg_chunk, None)
    for b in range(KB):
        jd = CH - KB + b
        pltpu.make_async_copy(ones_v, accs.at[dst_v.at[jd]], ssem).wait()
    plsc.subcore_barrier()

    pltpu.sync_copy(accs.at[pl.ds(row0, NR)], deg_hbm.at[pl.ds(row0, NR)])


def _sc_degree(dst3):
    mesh = plsc.VectorSubcoreMesh(**_SC_MESH)
    fn = functools.partial(
        pl.kernel, _sc_deg_body, mesh=mesh,
        compiler_params=pltpu.CompilerParams(use_tc_tiling_on_sc=False),
        out_type=jax.ShapeDtypeStruct((NP, R_P), jnp.float32),
        scratch_types=[
            pltpu.VMEM_SHARED((NP, R_P), jnp.float32),
            pltpu.VMEM((CH, 128), jnp.int32),
            pltpu.VMEM((NR, R_P), jnp.float32),
            pltpu.VMEM((128, R_P), jnp.float32),
            pltpu.SemaphoreType.DMA,
        ],
    )()
    zeros = jnp.zeros((NR, R_P), jnp.float32)
    ones = jnp.ones((128, R_P), jnp.float32)
    return fn(dst3, zeros, ones)


def _tc_pre_body(x_ref, b_ref, g_ref):
    g = jnp.dot(x_ref[...], b_ref[...], preferred_element_type=jnp.float32)
    for k in range(ORDER + 1):
        g_ref[k] = g[:, k * R_P:(k + 1) * R_P]


def _tc_pre(xpad, bprop):
    bn = 1024
    return pl.pallas_call(
        _tc_pre_body,
        grid=(NP // bn,),
        in_specs=[
            pl.BlockSpec((bn, IN_C), lambda i: (i, 0)),
            pl.BlockSpec((IN_C, (ORDER + 1) * R_P), lambda i: (0, 0)),
        ],
        out_specs=pl.BlockSpec((ORDER + 1, bn, R_P), lambda i: (0, i, 0)),
        out_shape=jax.ShapeDtypeStruct((ORDER + 1, NP, R_P), jnp.float32),
    )(xpad, bprop)


def _tc_dinv_body(deg_ref, dinv_ref, d2_ref):
    deg = deg_ref[...]
    dinv = jnp.where(deg > 0.0, lax.rsqrt(jnp.maximum(deg, 1.0)), 0.0)
    dinv_ref[...] = dinv
    d2_ref[...] = dinv * dinv


def _tc_dinv(degrow):
    return pl.pallas_call(
        _tc_dinv_body,
        out_shape=[
            jax.ShapeDtypeStruct((NP, R_P), jnp.float32),
            jax.ShapeDtypeStruct((NP, R_P), jnp.float32),
        ],
    )(degrow)


def _tc_post_body(t_ref, wp_ref, o_ref):
    o_ref[...] = jnp.dot(t_ref[...], wp_ref[...],
                         preferred_element_type=jnp.float32)


def _tc_post(t, W_P):
    bn = 1000
    return pl.pallas_call(
        _tc_post_body,
        grid=(N // bn,),
        in_specs=[
            pl.BlockSpec((bn, R_P), lambda i: (i, 0)),
            pl.BlockSpec((R_P, OUT_C), lambda i: (0, 0)),
        ],
        out_specs=pl.BlockSpec((bn, OUT_C), lambda i: (i, 0)),
        out_shape=jax.ShapeDtypeStruct((N, OUT_C), jnp.float32),
    )(t, W_P)


def _sc_hops_body(g_hbm, src_hbm, dst_hbm, d2_hbm, dinv_hbm, zeros_hbm,
                  t_hbm, w0_hbm, w1_hbm,
                  accs, src_v, dst_v, rows_v, acc_v, g_v,
                  dinv_v, d2_v, zero_v, gsem, ssem):
    sid = lax.axis_index("s")
    row0 = sid * NR

    pltpu.sync_copy(src_hbm.at[sid], src_v)
    pltpu.sync_copy(dst_hbm.at[sid], dst_v)
    pltpu.sync_copy(d2_hbm.at[pl.ds(row0, NR)], d2_v)
    pltpu.sync_copy(dinv_hbm.at[pl.ds(row0, NR)], dinv_v)
    pltpu.sync_copy(zeros_hbm, zero_v)

    pltpu.sync_copy(g_hbm.at[pl.ds(row0, NR)], g_v)

    def v0_row(r, carry):
        g_v[r] = dinv_v[r] * g_v[r]
        return carry

    lax.fori_loop(0, NR, v0_row, None)
    pltpu.sync_copy(g_v, w0_hbm.at[pl.ds(row0, NR)])
    pltpu.sync_copy(zero_v, accs.at[pl.ds(row0, NR)])
    plsc.subcore_barrier()

    wbufs = [w0_hbm, w1_hbm]
    for k in range(ORDER):
        w_nxt = wbufs[k % 2]
        w_gat = wbufs[0] if k == 0 else wbufs[(k - 1) % 2]

        for b in range(KB):
            pltpu.async_copy(w_gat.at[src_v.at[b]], rows_v.at[b], gsem)

        def edge_chunk(j, carry, w_gat=w_gat):
            slot = lax.rem(j, NS)
            pltpu.make_async_copy(w_gat.at[src_v.at[j]], rows_v.at[slot],
                                  gsem).wait()
            pltpu.async_copy(rows_v.at[slot], accs.at[dst_v.at[j]], ssem,
                             add=True)

            @pl.when(j >= KB)
            def _drain():
                jd = j - KB
                pltpu.make_async_copy(rows_v.at[lax.rem(jd, NS)],
                                      accs.at[dst_v.at[jd]], ssem).wait()

            @pl.when(j + KB < CH)
            def _prefetch():
                jg = j + KB
                pltpu.async_copy(w_gat.at[src_v.at[jg]],
                                 rows_v.at[lax.rem(jg, NS)], gsem)

            return carry

        lax.fori_loop(0, CH, edge_chunk, None)
        for b in range(KB):
            jd = CH - KB + b
            pltpu.make_async_copy(rows_v.at[jd % NS], accs.at[dst_v.at[jd]],
                                  ssem).wait()
        plsc.subcore_barrier()

        pltpu.sync_copy(accs.at[pl.ds(row0, NR)], acc_v)
        pltpu.sync_copy(zero_v, accs.at[pl.ds(row0, NR)])
        pltpu.sync_copy(g_hbm.at[pl.ds((k + 1) * NP + row0, NR)], g_v)

        out_hbm = w_nxt if k < ORDER - 1 else t_hbm

        if k < ORDER - 1:
            def upd_row(r, carry):
                g_v[r] = dinv_v[r] * g_v[r] + d2_v[r] * acc_v[r]
                return carry
        else:
            def upd_row(r, carry):
                g_v[r] = g_v[r] + dinv_v[r] * acc_v[r]
                return carry

        lax.fori_loop(0, NR, upd_row, None)
        pltpu.sync_copy(g_v, out_hbm.at[pl.ds(row0, NR)])
        plsc.subcore_barrier()


def _sc_propagate(gflat, src3, dst3, d2b, dinvb):
    mesh = plsc.VectorSubcoreMesh(**_SC_MESH)
    fn = functools.partial(
        pl.kernel, _sc_hops_body, mesh=mesh,
        compiler_params=pltpu.CompilerParams(use_tc_tiling_on_sc=False),
        out_type=[
            jax.ShapeDtypeStruct((NP, R_P), jnp.float32),
            jax.ShapeDtypeStruct((NP, R_P), jnp.float32),
            jax.ShapeDtypeStruct((NP, R_P), jnp.float32),
        ],
        scratch_types=[
            pltpu.VMEM_SHARED((NP, R_P), jnp.float32),
            pltpu.VMEM((CH, 128), jnp.int32),
            pltpu.VMEM((CH, 128), jnp.int32),
            pltpu.VMEM((NS, 128, R_P), jnp.float32),
            pltpu.VMEM((NR, R_P), jnp.float32),
            pltpu.VMEM((NR, R_P), jnp.float32),
            pltpu.VMEM((NR, R_P), jnp.float32),
            pltpu.VMEM((NR, R_P), jnp.float32),
            pltpu.VMEM((NR, R_P), jnp.float32),
            pltpu.SemaphoreType.DMA,
            pltpu.SemaphoreType.DMA,
        ],
    )()
    zeros = jnp.zeros((NR, R_P), jnp.float32)
    t, _, _ = fn(gflat, src3, dst3, d2b, dinvb, zeros)
    return t


def kernel(x, W_C, W_lamb, conv_w, comb_w, W_P, edge_index):
    a = (conv_w * jnp.tanh(1.0 / (conv_w + 1e-5)))[:, 0, :]
    c = jnp.cumprod(a, axis=0)
    comb = comb_w[0, :, 0, :]
    u = c * jnp.tile(comb, (1, R_P))
    kk = np.arange(RANK)
    sel = jnp.asarray((kk[:, None] // R_D) ==
                      np.arange(R_P)[None, :], jnp.float32)
    ublocks = [u[ORDER - k][:, None] * sel for k in range(ORDER + 1)]
    umat = jnp.concatenate(ublocks, axis=1)
    bprop = W_C @ W_lamb @ umat

    src = edge_index[0]
    dst = edge_index[1]
    pad = jnp.full((EP - E,), NP - 1, jnp.int32)
    src3 = jnp.concatenate([src, pad]).reshape(NT, CH, 128)
    dst3 = jnp.concatenate([dst, pad]).reshape(NT, CH, 128)

    degrow = _sc_degree(dst3)
    xpad = jnp.pad(x, ((0, NP - N), (0, 0)))
    gprop = _tc_pre(xpad, bprop)
    gflat = gprop.reshape((ORDER + 1) * NP, R_P)
    dinvb, d2b = _tc_dinv(degrow)

    t = _sc_propagate(gflat, src3, dst3, d2b, dinvb)
    return _tc_post(t[:N], W_P)


# speedup vs baseline: 34.6596x; 1.0255x over previous
import functools

import jax
import jax.numpy as jnp
import numpy as np
from jax import lax
from jax.experimental import pallas as pl
from jax.experimental.pallas import tpu as pltpu
from jax.experimental.pallas import tpu_sc as plsc

N = 10000
E = 320000
IN_C = 128
OUT_C = 128
R_D = 8
R_P = 16
ORDER = 10
RANK = R_D * R_P

NT = 16
NP = 10240
NR = NP // NT
CH = 157
EW = CH * 128
EP = NT * EW

_SC_MESH = dict(core_axis_name="c", subcore_axis_name="s", num_cores=1)

KB = 7
NS = 2 * KB


def _sc_deg_body(dst_hbm, zeros_hbm, ones_hbm, deg_hbm,
                 accs, dst_v, zero_v, ones_v, ssem):
    sid = lax.axis_index("s")
    row0 = sid * NR

    pltpu.sync_copy(dst_hbm.at[sid], dst_v)
    pltpu.sync_copy(zeros_hbm, zero_v)
    pltpu.sync_copy(ones_hbm, ones_v)

    pltpu.sync_copy(zero_v, accs.at[pl.ds(row0, NR)])
    plsc.subcore_barrier()

    def deg_chunk(j, carry):
        pltpu.async_copy(ones_v, accs.at[dst_v.at[j]], ssem, add=True)

        @pl.when(j >= KB)
        def _drain():
            pltpu.make_async_copy(ones_v, accs.at[dst_v.at[j - KB]],
                                  ssem).wait()

        return carry

    lax.fori_loop(0, CH, deg_chunk, None)
    for b in range(KB):
        jd = CH - KB + b
        pltpu.make_async_copy(ones_v, accs.at[dst_v.at[jd]], ssem).wait()
    plsc.subcore_barrier()

    pltpu.sync_copy(accs.at[pl.ds(row0, NR)], deg_hbm.at[pl.ds(row0, NR)])


def _sc_degree(dst3):
    mesh = plsc.VectorSubcoreMesh(**_SC_MESH)
    fn = functools.partial(
        pl.kernel, _sc_deg_body, mesh=mesh,
        compiler_params=pltpu.CompilerParams(use_tc_tiling_on_sc=False),
        out_type=jax.ShapeDtypeStruct((NP, R_P), jnp.float32),
        scratch_types=[
            pltpu.VMEM_SHARED((NP, R_P), jnp.float32),
            pltpu.VMEM((CH, 128), jnp.int32),
            pltpu.VMEM((NR, R_P), jnp.float32),
            pltpu.VMEM((128, R_P), jnp.float32),
            pltpu.SemaphoreType.DMA,
        ],
    )()
    zeros = jnp.zeros((NR, R_P), jnp.float32)
    ones = jnp.ones((128, R_P), jnp.float32)
    return fn(dst3, zeros, ones)


def _tc_pre_body(x_ref, b_ref, g_ref):
    g = jnp.dot(x_ref[...], b_ref[...], preferred_element_type=jnp.float32)
    for k in range(ORDER + 1):
        g_ref[k] = g[:, k * R_P:(k + 1) * R_P]


def _tc_pre(xpad, bprop):
    bn = 1024
    return pl.pallas_call(
        _tc_pre_body,
        grid=(NP // bn,),
        in_specs=[
            pl.BlockSpec((bn, IN_C), lambda i: (i, 0)),
            pl.BlockSpec((IN_C, (ORDER + 1) * R_P), lambda i: (0, 0)),
        ],
        out_specs=pl.BlockSpec((ORDER + 1, bn, R_P), lambda i: (0, i, 0)),
        out_shape=jax.ShapeDtypeStruct((ORDER + 1, NP, R_P), jnp.float32),
    )(xpad, bprop)


def _tc_dinv_body(deg_ref, dinv_ref, d2_ref):
    deg = deg_ref[...]
    dinv = jnp.where(deg > 0.0, lax.rsqrt(jnp.maximum(deg, 1.0)), 0.0)
    dinv_ref[...] = dinv
    d2_ref[...] = dinv * dinv


def _tc_dinv(degrow):
    return pl.pallas_call(
        _tc_dinv_body,
        out_shape=[
            jax.ShapeDtypeStruct((NP, R_P), jnp.float32),
            jax.ShapeDtypeStruct((NP, R_P), jnp.float32),
        ],
    )(degrow)


def _tc_post_body(t_ref, wp_ref, o_ref):
    o_ref[...] = jnp.dot(t_ref[...], wp_ref[...],
                         preferred_element_type=jnp.float32)


def _tc_post(t, W_P):
    bn = 1000
    return pl.pallas_call(
        _tc_post_body,
        grid=(N // bn,),
        in_specs=[
            pl.BlockSpec((bn, R_P), lambda i: (i, 0)),
            pl.BlockSpec((R_P, OUT_C), lambda i: (0, 0)),
        ],
        out_specs=pl.BlockSpec((bn, OUT_C), lambda i: (i, 0)),
        out_shape=jax.ShapeDtypeStruct((N, OUT_C), jnp.float32),
    )(t, W_P)


def _sc_hops_body(g_hbm, src_hbm, dst_hbm, d2_hbm, dinv_hbm, zeros_hbm,
                  t_hbm, w0_hbm, w1_hbm,
                  accs, src_v, dst_v, rows_v, acc_v, g_v,
                  dinv_v, d2_v, zero_v, gsem, ssem, g2sem, zsem):
    sid = lax.axis_index("s")
    row0 = sid * NR

    pltpu.sync_copy(src_hbm.at[sid], src_v)
    pltpu.sync_copy(dst_hbm.at[sid], dst_v)
    pltpu.sync_copy(d2_hbm.at[pl.ds(row0, NR)], d2_v)
    pltpu.sync_copy(dinv_hbm.at[pl.ds(row0, NR)], dinv_v)
    pltpu.sync_copy(zeros_hbm, zero_v)

    pltpu.sync_copy(g_hbm.at[pl.ds(row0, NR)], g_v)

    def v0_row(r, carry):
        g_v[r] = dinv_v[r] * g_v[r]
        return carry

    lax.fori_loop(0, NR, v0_row, None)
    pltpu.sync_copy(g_v, w0_hbm.at[pl.ds(row0, NR)])
    pltpu.sync_copy(zero_v, accs.at[pl.ds(row0, NR)])
    plsc.subcore_barrier()

    wbufs = [w0_hbm, w1_hbm]
    for k in range(ORDER):
        w_nxt = wbufs[k % 2]
        w_gat = wbufs[0] if k == 0 else wbufs[(k - 1) % 2]

        pltpu.async_copy(g_hbm.at[pl.ds((k + 1) * NP + row0, NR)], g_v,
                         g2sem)

        for b in range(KB):
            pltpu.async_copy(w_gat.at[src_v.at[b]], rows_v.at[b], gsem)

        def edge_chunk(j, carry, w_gat=w_gat):
            slot = lax.rem(j, NS)
            pltpu.make_async_copy(w_gat.at[src_v.at[j]], rows_v.at[slot],
                                  gsem).wait()
            pltpu.async_copy(rows_v.at[slot], accs.at[dst_v.at[j]], ssem,
                             add=True)

            @pl.when(j >= KB)
            def _drain():
                jd = j - KB
                pltpu.make_async_copy(rows_v.at[lax.rem(jd, NS)],
                                      accs.at[dst_v.at[jd]], ssem).wait()

            @pl.when(j + KB < CH)
            def _prefetch():
                jg = j + KB
                pltpu.async_copy(w_gat.at[src_v.at[jg]],
                                 rows_v.at[lax.rem(jg, NS)], gsem)

            return carry

        lax.fori_loop(0, CH, edge_chunk, None)
        for b in range(KB):
            jd = CH - KB + b
            pltpu.make_async_copy(rows_v.at[jd % NS], accs.at[dst_v.at[jd]],
                                  ssem).wait()
        plsc.subcore_barrier()

        pltpu.sync_copy(accs.at[pl.ds(row0, NR)], acc_v)
        pltpu.async_copy(zero_v, accs.at[pl.ds(row0, NR)], zsem)
        pltpu.make_async_copy(g_hbm.at[pl.ds((k + 1) * NP + row0, NR)], g_v,
                              g2sem).wait()

        out_hbm = w_nxt if k < ORDER - 1 else t_hbm

        if k < ORDER - 1:
            def upd_row(r, carry):
                g_v[r] = dinv_v[r] * g_v[r] + d2_v[r] * acc_v[r]
                return carry
        else:
            def upd_row(r, carry):
                g_v[r] = g_v[r] + dinv_v[r] * acc_v[r]
                return carry

        lax.fori_loop(0, NR, upd_row, None)
        pltpu.sync_copy(g_v, out_hbm.at[pl.ds(row0, NR)])
        pltpu.make_async_copy(zero_v, accs.at[pl.ds(row0, NR)], zsem).wait()
        plsc.subcore_barrier()


def _sc_propagate(gflat, src3, dst3, d2b, dinvb):
    mesh = plsc.VectorSubcoreMesh(**_SC_MESH)
    fn = functools.partial(
        pl.kernel, _sc_hops_body, mesh=mesh,
        compiler_params=pltpu.CompilerParams(use_tc_tiling_on_sc=False),
        out_type=[
            jax.ShapeDtypeStruct((NP, R_P), jnp.float32),
            jax.ShapeDtypeStruct((NP, R_P), jnp.float32),
            jax.ShapeDtypeStruct((NP, R_P), jnp.float32),
        ],
        scratch_types=[
            pltpu.VMEM_SHARED((NP, R_P), jnp.float32),
            pltpu.VMEM((CH, 128), jnp.int32),
            pltpu.VMEM((CH, 128), jnp.int32),
            pltpu.VMEM((NS, 128, R_P), jnp.float32),
            pltpu.VMEM((NR, R_P), jnp.float32),
            pltpu.VMEM((NR, R_P), jnp.float32),
            pltpu.VMEM((NR, R_P), jnp.float32),
            pltpu.VMEM((NR, R_P), jnp.float32),
            pltpu.VMEM((NR, R_P), jnp.float32),
            pltpu.SemaphoreType.DMA,
            pltpu.SemaphoreType.DMA,
            pltpu.SemaphoreType.DMA,
            pltpu.SemaphoreType.DMA,
        ],
    )()
    zeros = jnp.zeros((NR, R_P), jnp.float32)
    t, _, _ = fn(gflat, src3, dst3, d2b, dinvb, zeros)
    return t


def kernel(x, W_C, W_lamb, conv_w, comb_w, W_P, edge_index):
    a = (conv_w * jnp.tanh(1.0 / (conv_w + 1e-5)))[:, 0, :]
    c = jnp.cumprod(a, axis=0)
    comb = comb_w[0, :, 0, :]
    u = c * jnp.tile(comb, (1, R_P))
    kk = np.arange(RANK)
    sel = jnp.asarray((kk[:, None] // R_D) ==
                      np.arange(R_P)[None, :], jnp.float32)
    ublocks = [u[ORDER - k][:, None] * sel for k in range(ORDER + 1)]
    umat = jnp.concatenate(ublocks, axis=1)
    bprop = W_C @ W_lamb @ umat

    src = edge_index[0]
    dst = edge_index[1]
    pad = jnp.full((EP - E,), NP - 1, jnp.int32)
    src3 = jnp.concatenate([src, pad]).reshape(NT, CH, 128)
    dst3 = jnp.concatenate([dst, pad]).reshape(NT, CH, 128)

    degrow = _sc_degree(dst3)
    xpad = jnp.pad(x, ((0, NP - N), (0, 0)))
    gprop = _tc_pre(xpad, bprop)
    gflat = gprop.reshape((ORDER + 1) * NP, R_P)
    dinvb, d2b = _tc_dinv(degrow)

    t = _sc_propagate(gflat, src3, dst3, d2b, dinvb)
    return _tc_post(t[:N], W_P)
